# baseline (speedup 1.0000x reference)
import functools

import jax
import jax.numpy as jnp
from jax import lax
from jax.experimental import pallas as pl
from jax.experimental.pallas import tpu as pltpu
from jax.experimental.pallas import tpu_sc as plsc

_NC = 2
_NS = 16
_NW = _NC * _NS
_CHROWS = 128


def _gather_body(tbl_hbm, idx_hbm, out_hbm, idx_v, rows_v, tbl_sp, gsem):
    n_ch = idx_hbm.shape[1]
    wid = lax.axis_index("s") * _NC + lax.axis_index("c")

    @pl.when(lax.axis_index("s") == 0)
    def _():
        pltpu.sync_copy(tbl_hbm, tbl_sp)

    pltpu.sync_copy(idx_hbm.at[wid], idx_v)
    plsc.subcore_barrier()
    base = wid * (n_ch * _CHROWS)
    pltpu.make_async_copy(tbl_sp.at[idx_v.at[0]], rows_v.at[0], gsem).start()

    def body(c, carry):
        @pl.when(c + 1 < n_ch)
        def _():
            pltpu.make_async_copy(
                tbl_sp.at[idx_v.at[c + 1]], rows_v.at[(c + 1) % 2], gsem
            ).start()

        pltpu.make_async_copy(
            tbl_sp.at[idx_v.at[c]], rows_v.at[c % 2], gsem
        ).wait()
        pltpu.sync_copy(
            rows_v.at[c % 2], out_hbm.at[pl.ds(base + c * _CHROWS, _CHROWS)]
        )
        return carry

    lax.fori_loop(0, n_ch, body, 0)


def _sc_gather(table, flat_idx):
    n, d = table.shape
    e = flat_idx.shape[0]
    n_ch = e // (_NW * _CHROWS)
    idx3 = flat_idx.reshape(_NW, n_ch, _CHROWS)
    mesh = plsc.VectorSubcoreMesh(
        core_axis_name="c", subcore_axis_name="s", num_cores=_NC, num_subcores=_NS
    )
    run = pl.kernel(
        _gather_body,
        out_type=jax.ShapeDtypeStruct((e, d), table.dtype),
        mesh=mesh,
        scratch_types=[
            pltpu.VMEM((n_ch, _CHROWS), jnp.int32),
            pltpu.VMEM((2, _CHROWS, d), table.dtype),
            pltpu.VMEM_SHARED((n, d), table.dtype),
            pltpu.SemaphoreType.DMA,
        ],
    )
    return run(table, idx3)


def _proj_body(node_ref, w1b_ref, p_ref):
    p_ref[...] = jnp.dot(node_ref[...], w1b_ref[...],
                         preferred_element_type=jnp.float32)


def _tc_proj(node_flat, w1b):
    n, d = node_flat.shape
    return pl.pallas_call(
        _proj_body,
        out_shape=jax.ShapeDtypeStruct((n, d), jnp.float32),
    )(node_flat, w1b)


def _rbf_proj_body(x_ref, w1d_ref, out_ref):
    kb = x_ref.shape[1]
    rb = out_ref.shape[1]
    br = out_ref.shape[3]
    f32, bf16 = jnp.float32, jnp.bfloat16
    w1d = w1d_ref[...].astype(bf16)
    for kk in range(kb):
        y = jax.lax.dot_general(
            x_ref[0, kk].astype(bf16), w1d,
            (((0,), (0,)), ((), ())), preferred_element_type=f32)
        for jj in range(rb):
            out_ref[0, jj, kk] = y[jj * br:(jj + 1) * br].astype(bf16)


def _tc_rbf_proj(rbf_t, w1d, br):
    b, k, d_rbf, r = rbf_t.shape
    d = w1d.shape[1]
    kb = 32
    rb = r // br
    return pl.pallas_call(
        _rbf_proj_body,
        grid=(b, k // kb),
        in_specs=[
            pl.BlockSpec((1, kb, d_rbf, r), lambda i, j: (i, j, 0, 0)),
            pl.BlockSpec((d_rbf, d), lambda i, j: (0, 0)),
        ],
        out_specs=pl.BlockSpec((1, rb, kb, br, d), lambda i, j: (i, 0, j, 0, 0)),
        out_shape=jax.ShapeDtypeStruct((b, rb, k, br, d), jnp.bfloat16),
    )(rbf_t, w1d)


def _mlp_body(node_ref, g_ref, e_ref, rbfc_ref, w1_ref, w2_ref, out_ref):
    br = node_ref.shape[1]
    k = e_ref.shape[2]
    d = node_ref.shape[2]
    f32, bf16 = jnp.float32, jnp.bfloat16
    w1 = w1_ref[...].astype(bf16)
    a = jnp.dot(node_ref[0].astype(bf16), w1[:d],
                preferred_element_type=f32)
    ge = jnp.concatenate(
        [g_ref[0].reshape(br * k, d).astype(bf16),
         e_ref[0].reshape(br * k, d).astype(bf16)], axis=1)
    hid = jnp.dot(ge, w1[d:3 * d], preferred_element_type=f32)
    rc = jnp.swapaxes(rbfc_ref[0, 0], 0, 1)
    hid += rc.reshape(br * k, d).astype(f32)
    hid = (hid.reshape(br, k, d) + a[:, None, :]).reshape(br * k, d)
    out = jnp.dot(hid.astype(bf16), w2_ref[...].astype(bf16),
                  preferred_element_type=f32)
    out_ref[0] = (out * jax.nn.sigmoid(out)).reshape(br, k, d)


def _tc_mlp(node_embed, gathered, local_edge_embed, rbfc_t, w1, w2):
    b, r, d = node_embed.shape
    k = local_edge_embed.shape[2]
    br = 256
    g4 = gathered.reshape(b, r, k, d)
    grid = (b, r // br)
    full = lambda shape: pl.BlockSpec(shape, lambda i, j: (0,) * len(shape))
    return pl.pallas_call(
        _mlp_body,
        grid=grid,
        in_specs=[
            pl.BlockSpec((1, br, d), lambda i, j: (i, j, 0)),
            pl.BlockSpec((1, br, k, d), lambda i, j: (i, j, 0, 0)),
            pl.BlockSpec((1, br, k, d), lambda i, j: (i, j, 0, 0)),
            pl.BlockSpec((1, 1, k, br, d), lambda i, j: (i, j, 0, 0, 0)),
            full(w1.shape),
            full((d, d)),
        ],
        out_specs=pl.BlockSpec((1, br, k, d), lambda i, j: (i, j, 0, 0)),
        out_shape=jax.ShapeDtypeStruct((b, r, k, d), jnp.float32),
    )(node_embed, g4, local_edge_embed, rbfc_t, w1, w2)


def kernel(node_embed, local_edge_embed, rbf_embed, local_graph, W1, W2):
    b, r, d = node_embed.shape
    k = local_edge_embed.shape[2]
    flat_idx = (jnp.arange(b, dtype=jnp.int32)[:, None, None] * r
                + local_graph.astype(jnp.int32)).reshape(-1)
    rbf_t = jnp.transpose(rbf_embed, (0, 2, 3, 1))
    rbfc_t = _tc_rbf_proj(rbf_t, W1[3 * d:], 256)
    gathered = _sc_gather(node_embed.reshape(b * r, d), flat_idx)
    return _tc_mlp(node_embed, gathered, local_edge_embed, rbfc_t, W1, W2)


# speedup vs baseline: 1.0115x; 1.0115x over previous
import jax
import jax.numpy as jnp
from jax import lax
from jax.experimental import pallas as pl
from jax.experimental.pallas import tpu as pltpu
from jax.experimental.pallas import tpu_sc as plsc

_NC = 2
_NS = 16
_NW = _NC * _NS
_CHROWS = 128


def _gather_body(tbl_hbm, idx_hbm, out_hbm, idx_v, rows_v, tbl_sp, gsem):
    n_ch = idx_hbm.shape[1]
    wid = lax.axis_index("s") * _NC + lax.axis_index("c")

    @pl.when(lax.axis_index("s") == 0)
    def _():
        pltpu.sync_copy(tbl_hbm, tbl_sp)

    pltpu.sync_copy(idx_hbm.at[wid], idx_v)
    plsc.subcore_barrier()
    base = wid * (n_ch * _CHROWS)
    pltpu.make_async_copy(tbl_sp.at[idx_v.at[0]], rows_v.at[0], gsem).start()

    def body(c, carry):
        @pl.when(c + 1 < n_ch)
        def _():
            pltpu.make_async_copy(
                tbl_sp.at[idx_v.at[c + 1]], rows_v.at[(c + 1) % 2], gsem
            ).start()

        pltpu.make_async_copy(
            tbl_sp.at[idx_v.at[c]], rows_v.at[c % 2], gsem
        ).wait()
        pltpu.sync_copy(
            rows_v.at[c % 2], out_hbm.at[pl.ds(base + c * _CHROWS, _CHROWS)]
        )
        return carry

    lax.fori_loop(0, n_ch, body, 0)


def _sc_gather(table, flat_idx):
    n, d = table.shape
    e = flat_idx.shape[0]
    n_ch = e // (_NW * _CHROWS)
    idx3 = flat_idx.reshape(_NW, n_ch, _CHROWS)
    mesh = plsc.VectorSubcoreMesh(
        core_axis_name="c", subcore_axis_name="s", num_cores=_NC, num_subcores=_NS
    )
    run = pl.kernel(
        _gather_body,
        out_type=jax.ShapeDtypeStruct((e, d), table.dtype),
        mesh=mesh,
        scratch_types=[
            pltpu.VMEM((n_ch, _CHROWS), jnp.int32),
            pltpu.VMEM((2, _CHROWS, d), table.dtype),
            pltpu.VMEM_SHARED((n, d), table.dtype),
            pltpu.SemaphoreType.DMA,
        ],
    )
    return run(table, idx3)


def _rbf_proj_body(x_ref, w1d_ref, out_ref):
    kb = x_ref.shape[1]
    f32, bf16 = jnp.float32, jnp.bfloat16
    w1d = w1d_ref[...].astype(bf16)
    for kk in range(kb):
        y = jax.lax.dot_general(
            x_ref[0, kk].astype(bf16), w1d,
            (((0,), (0,)), ((), ())), preferred_element_type=f32)
        out_ref[0, kk] = y.astype(bf16)


def _tc_rbf_proj(rbf_t, w1d):
    b, k, d_rbf, r = rbf_t.shape
    d = w1d.shape[1]
    kb = 32
    return pl.pallas_call(
        _rbf_proj_body,
        grid=(b, k // kb),
        in_specs=[
            pl.BlockSpec((1, kb, d_rbf, r), lambda i, j: (i, j, 0, 0)),
            pl.BlockSpec((d_rbf, d), lambda i, j: (0, 0)),
        ],
        out_specs=pl.BlockSpec((1, kb, r, d), lambda i, j: (i, j, 0, 0)),
        out_shape=jax.ShapeDtypeStruct((b, k, r, d), jnp.bfloat16),
    )(rbf_t, w1d)


def _mlp_body(node_ref, g_ref, e_ref, rbfc_ref, w1_ref, w2_ref, out_ref):
    br = node_ref.shape[1]
    k = e_ref.shape[2]
    d = node_ref.shape[2]
    f32, bf16 = jnp.float32, jnp.bfloat16
    w1 = w1_ref[...].astype(bf16)
    a = jnp.dot(node_ref[0].astype(bf16), w1[:d],
                preferred_element_type=f32)
    ge = jnp.concatenate(
        [g_ref[0].reshape(br * k, d).astype(bf16),
         e_ref[0].reshape(br * k, d).astype(bf16)], axis=1)
    hid = jnp.dot(ge, w1[d:3 * d], preferred_element_type=f32)
    rc = jnp.swapaxes(rbfc_ref[0], 0, 1)
    hid += rc.reshape(br * k, d).astype(f32)
    hid = (hid.reshape(br, k, d) + a[:, None, :]).reshape(br * k, d)
    out = jnp.dot(hid.astype(bf16), w2_ref[...].astype(bf16),
                  preferred_element_type=f32)
    out_ref[0] = (out * jax.nn.sigmoid(out)).reshape(br, k, d)


def _tc_mlp(node_embed, gathered, local_edge_embed, rbfc_t, w1, w2):
    b, r, d = node_embed.shape
    k = local_edge_embed.shape[2]
    br = 256
    g4 = gathered.reshape(b, r, k, d)
    grid = (b, r // br)
    full = lambda shape: pl.BlockSpec(shape, lambda i, j: (0,) * len(shape))
    return pl.pallas_call(
        _mlp_body,
        grid=grid,
        in_specs=[
            pl.BlockSpec((1, br, d), lambda i, j: (i, j, 0)),
            pl.BlockSpec((1, br, k, d), lambda i, j: (i, j, 0, 0)),
            pl.BlockSpec((1, br, k, d), lambda i, j: (i, j, 0, 0)),
            pl.BlockSpec((1, k, br, d), lambda i, j: (i, 0, j, 0)),
            full(w1.shape),
            full((d, d)),
        ],
        out_specs=pl.BlockSpec((1, br, k, d), lambda i, j: (i, j, 0, 0)),
        out_shape=jax.ShapeDtypeStruct((b, r, k, d), jnp.float32),
    )(node_embed, g4, local_edge_embed, rbfc_t, w1, w2)


def kernel(node_embed, local_edge_embed, rbf_embed, local_graph, W1, W2):
    b, r, d = node_embed.shape
    k = local_edge_embed.shape[2]
    flat_idx = (jnp.arange(b, dtype=jnp.int32)[:, None, None] * r
                + local_graph.astype(jnp.int32)).reshape(-1)
    rbf_t = jnp.transpose(rbf_embed, (0, 2, 3, 1))
    rbfc_t = _tc_rbf_proj(rbf_t, W1[3 * d:])
    gathered = _sc_gather(node_embed.reshape(b * r, d), flat_idx)
    return _tc_mlp(node_embed, gathered, local_edge_embed, rbfc_t, W1, W2)
